# baseline (speedup 1.0000x reference)
import functools

import jax
import jax.numpy as jnp
from jax import lax
from jax.experimental import pallas as pl
from jax.experimental.pallas import tpu as pltpu
from jax.experimental.pallas import tpu_sc as plsc

_TOPK = 2
_E = 16


def _probs_body(x_ref, w_ref, b_ref, p_ref):
    logits = lax.dot_general(
        x_ref[...], w_ref[...], (((1,), (1,)), ((), ())),
        preferred_element_type=jnp.float32) + b_ref[...]
    p_ref[...] = 1.0 / (1.0 + jnp.exp(-logits))


def _make_sc_router(n):
    info = plsc.get_sparse_core_info()
    nw = info.num_cores * info.num_subcores
    rows = n // nw
    mesh = plsc.VectorSubcoreMesh(core_axis_name="c", subcore_axis_name="s")

    @functools.partial(
        pl.kernel,
        out_type=(
            jax.ShapeDtypeStruct((n * _TOPK,), jnp.float32),
            jax.ShapeDtypeStruct((n * _TOPK,), jnp.int32),
            jax.ShapeDtypeStruct((n * _E,), jnp.float32),
        ),
        mesh=mesh,
        scratch_types=[
            pltpu.VMEM((rows * _E,), jnp.float32),
            pltpu.VMEM((rows * _TOPK,), jnp.float32),
            pltpu.VMEM((rows * _TOPK,), jnp.int32),
            pltpu.VMEM((rows * _E,), jnp.float32),
        ],
        compiler_params=pltpu.CompilerParams(needs_layout_passes=False),
    )
    def sc_router(p_hbm, topw_hbm, topi_hbm, rw_hbm, p_v, tw_v, ti_v, rw_v):
        wid = lax.axis_index("s") * info.num_cores + lax.axis_index("c")
        base = wid * rows
        pltpu.sync_copy(p_hbm.at[pl.ds(base * _E, rows * _E)], p_v)
        lane = lax.iota(jnp.int32, 16)
        mask2 = lane < _TOPK
        zero = jnp.zeros((_E,), jnp.float32)

        def body(r, carry):
            row = p_v[pl.ds(r * _E, _E)]
            s, v = plsc.sort_key_val(row, lane, descending=True)
            rw_v[pl.ds(r * _E, _E)] = zero
            plsc.store_scatter(rw_v, [v + r * _E], s, mask=mask2)
            plsc.store_scatter(tw_v, [lane + r * _TOPK], s, mask=mask2)
            plsc.store_scatter(ti_v, [lane + r * _TOPK], v, mask=mask2)
            return carry

        lax.fori_loop(0, rows, body, 0)
        pltpu.sync_copy(tw_v, topw_hbm.at[pl.ds(base * _TOPK, rows * _TOPK)])
        pltpu.sync_copy(ti_v, topi_hbm.at[pl.ds(base * _TOPK, rows * _TOPK)])
        pltpu.sync_copy(rw_v, rw_hbm.at[pl.ds(base * _E, rows * _E)])

    return sc_router


def kernel(x, W, b):
    batch, seq, d = x.shape
    n = batch * seq
    xf = x.reshape(n, d)
    bt = 1024
    probs = pl.pallas_call(
        _probs_body,
        grid=(n // bt,),
        in_specs=[
            pl.BlockSpec((bt, d), lambda i: (i, 0)),
            pl.BlockSpec((_E, d), lambda i: (0, 0)),
            pl.BlockSpec((1, _E), lambda i: (0, 0)),
        ],
        out_specs=pl.BlockSpec((bt, _E), lambda i: (i, 0)),
        out_shape=jax.ShapeDtypeStruct((n, _E), jnp.float32),
    )(xf, W, b.reshape(1, _E))
    topw, topi, rw = _make_sc_router(n)(probs.reshape(-1))
    return topw.reshape(n, _TOPK), topi.reshape(n, _TOPK), rw.reshape(n, _E)


# speedup vs baseline: 1.7447x; 1.7447x over previous
import jax
import jax.numpy as jnp
from jax import lax
from jax.experimental import pallas as pl

_TOPK = 2
_E = 16


def _router_body(x_ref, w_ref, b_ref, topw_ref, topi_ref, rw_ref):
    xb = x_ref[...]
    w = w_ref[...]
    b = b_ref[...]
    logits = lax.dot_general(
        xb, w, (((1,), (1,)), ((), ())),
        preferred_element_type=jnp.float32) + b
    probs = 1.0 / (1.0 + jnp.exp(-logits))
    eidx = lax.broadcasted_iota(jnp.int32, probs.shape, 1).astype(jnp.float32)
    max1 = jnp.max(probs, axis=1, keepdims=True)
    idx1 = jnp.min(jnp.where(probs == max1, eidx, float(_E)), axis=1,
                   keepdims=True)
    probs_m = jnp.where(eidx == idx1, -jnp.inf, probs)
    max2 = jnp.max(probs_m, axis=1, keepdims=True)
    idx2 = jnp.min(jnp.where(probs_m == max2, eidx, float(_E)), axis=1,
                   keepdims=True)
    topw_ref[...] = jnp.concatenate([max1, max2], axis=1)
    topi_ref[...] = jnp.concatenate([idx1, idx2], axis=1).astype(jnp.int32)
    keep = (eidx == idx1) | (eidx == idx2)
    rw_ref[...] = jnp.where(keep, probs, 0.0)


def kernel(x, W, b):
    batch, seq, d = x.shape
    n = batch * seq
    xf = x.reshape(n, d)
    bt = 2048
    grid = (n // bt,)
    out_shapes = (
        jax.ShapeDtypeStruct((n, _TOPK), jnp.float32),
        jax.ShapeDtypeStruct((n, _TOPK), jnp.int32),
        jax.ShapeDtypeStruct((n, _E), jnp.float32),
    )
    topw, topi, rw = pl.pallas_call(
        _router_body,
        grid=grid,
        in_specs=[
            pl.BlockSpec((bt, d), lambda i: (i, 0)),
            pl.BlockSpec((_E, d), lambda i: (0, 0)),
            pl.BlockSpec((1, _E), lambda i: (0, 0)),
        ],
        out_specs=[
            pl.BlockSpec((bt, _TOPK), lambda i: (i, 0)),
            pl.BlockSpec((bt, _TOPK), lambda i: (i, 0)),
            pl.BlockSpec((bt, _E), lambda i: (i, 0)),
        ],
        out_shape=out_shapes,
    )(xf, W, b.reshape(1, _E))
    return topw, topi, rw
